# Initial kernel scaffold; baseline (speedup 1.0000x reference)
import jax
import jax.numpy as jnp
from jax.experimental import pallas as pl


def kernel(feature, edge_index, W, b):
    raise NotImplementedError("write your pallas kernel here")


# speedup vs baseline: 2.9924x; 2.9924x over previous
import functools

import jax
import jax.numpy as jnp
from jax import lax
from jax.experimental import pallas as pl
from jax.experimental.pallas import tpu as pltpu
from jax.experimental.pallas import tpu_sc as plsc

N_NODES = 10000
N_EDGES = 320000
D = 128

NC = 2
NS = 16
NW = NC * NS
CHUNK = 128
K = 80
EP = NW * K * CHUNK
ACC_ROWS = 10112
RPW = ACC_ROWS // NS
LAST = N_NODES - (NS - 1) * RPW

_sc_mesh = plsc.VectorSubcoreMesh(core_axis_name="c", subcore_axis_name="s")


@functools.partial(
    pl.kernel,
    out_type=jax.ShapeDtypeStruct((NC, N_NODES, D), jnp.float32),
    mesh=_sc_mesh,
    scratch_types=[
        pltpu.MemorySpace.VMEM_SHARED((ACC_ROWS, D), jnp.float32),
        pltpu.VMEM((K, CHUNK), jnp.int32),
        pltpu.VMEM((K, CHUNK), jnp.int32),
        pltpu.VMEM((CHUNK, D), jnp.float32),
        pltpu.SemaphoreType.DMA,
    ],
)
def _sc_aggregate(feature_hbm, src_hbm, dst_hbm, zero_hbm, out_hbm,
                  acc, src_v, dst_v, rows, gsem):
    c = lax.axis_index("c")
    s = lax.axis_index("s")
    wid = c * NS + s

    pltpu.sync_copy(zero_hbm.at[pl.ds(s * RPW, RPW)], acc.at[pl.ds(s * RPW, RPW)])
    pltpu.sync_copy(src_hbm.at[pl.ds(wid * K, K)], src_v)
    pltpu.sync_copy(dst_hbm.at[pl.ds(wid * K, K)], dst_v)
    plsc.subcore_barrier()

    def chunk_body(j, carry):
        pltpu.async_copy(feature_hbm.at[src_v.at[j]], rows, gsem).wait()
        pltpu.sync_copy(rows, acc.at[dst_v.at[j]], add=True)
        return carry

    lax.fori_loop(0, K, chunk_body, 0)
    plsc.subcore_barrier()

    @pl.when(s < NS - 1)
    def _():
        pltpu.sync_copy(acc.at[pl.ds(s * RPW, RPW)],
                        out_hbm.at[c, pl.ds(s * RPW, RPW)])

    @pl.when(s == NS - 1)
    def _():
        pltpu.sync_copy(acc.at[pl.ds((NS - 1) * RPW, LAST)],
                        out_hbm.at[c, pl.ds((NS - 1) * RPW, LAST)])


def _tc_linear_body(p_ref, w_ref, b_ref, o_ref):
    x = p_ref[0] + p_ref[1]
    y = lax.dot_general(x, w_ref[...], (((1,), (1,)), ((), ())),
                        preferred_element_type=jnp.float32)
    o_ref[...] = y + b_ref[0:1, :]


def _tc_linear(parts, W, b8):
    M = 1000
    return pl.pallas_call(
        _tc_linear_body,
        grid=(N_NODES // M,),
        in_specs=[
            pl.BlockSpec((NC, M, D), lambda i: (0, i, 0)),
            pl.BlockSpec((D, D), lambda i: (0, 0)),
            pl.BlockSpec((8, D), lambda i: (0, 0)),
        ],
        out_specs=pl.BlockSpec((M, D), lambda i: (i, 0)),
        out_shape=jax.ShapeDtypeStruct((N_NODES, D), jnp.float32),
    )(parts, W, b8)


def kernel(feature, edge_index, W, b):
    src = edge_index[0].astype(jnp.int32)
    dst = edge_index[1].astype(jnp.int32)
    pad = EP - N_EDGES
    src_p = jnp.concatenate([src, jnp.zeros((pad,), jnp.int32)]).reshape(NW * K, CHUNK)
    dst_p = jnp.concatenate([dst, jnp.full((pad,), N_NODES, jnp.int32)]).reshape(NW * K, CHUNK)
    zeros = jnp.zeros((ACC_ROWS, D), jnp.float32)
    parts = _sc_aggregate(feature, src_p, dst_p, zeros)
    return _tc_linear(parts, W, jnp.broadcast_to(b, (8, D)))
